# Initial kernel scaffold; baseline (speedup 1.0000x reference)
import jax
import jax.numpy as jnp
from jax import lax
from jax.experimental import pallas as pl
from jax.experimental.pallas import tpu as pltpu
from jax.experimental.pallas import tpu_sc as plsc

N_EMB = 8192
DIM = 256
M_TOK = 16384
BM = 512
BN = 2048
GM = M_TOK // BM

_NC = 2
_NS = 16
_NW = _NC * _NS
_BW = M_TOK // _NW
_CH = 128


def _argmin_body(x_ref, e_ref, a_ref, b_ref, idx_ref, dsum_ref):
    i = pl.program_id(0)
    x = x_ref[...]
    a = a_ref[...]
    run_min = None
    run_idx = None
    for c in range(N_EMB // BN):
        ec = e_ref[pl.ds(c * BN, BN), :]
        bc = b_ref[:, pl.ds(c * BN, BN)]
        m = lax.dot_general(x, ec, (((1,), (1,)), ((), ())),
                            preferred_element_type=jnp.float32)
        d = (a + bc) - 2.0 * m
        cmin = jnp.min(d, axis=1, keepdims=True)
        col = lax.broadcasted_iota(jnp.int32, (BM, BN), 1) + c * BN
        cidx = jnp.min(jnp.where(d == cmin, col, jnp.int32(N_EMB)),
                       axis=1, keepdims=True)
        if run_min is None:
            run_min, run_idx = cmin, cidx
        else:
            better = cmin < run_min
            run_idx = jnp.where(better, cidx, run_idx)
            run_min = jnp.where(better, cmin, run_min)
    idx_ref[...] = run_idx

    @pl.when(i == 0)
    def _():
        dsum_ref[...] = jnp.zeros((1, 1), jnp.float32)

    dsum_ref[...] += jnp.sum(run_min, axis=0, keepdims=True)


def _dist_argmin(x, e, a, b):
    return pl.pallas_call(
        _argmin_body,
        grid=(GM,),
        in_specs=[
            pl.BlockSpec((BM, DIM), lambda i: (i, 0)),
            pl.BlockSpec((N_EMB, DIM), lambda i: (0, 0)),
            pl.BlockSpec((BM, 1), lambda i: (i, 0)),
            pl.BlockSpec((1, N_EMB), lambda i: (0, 0)),
        ],
        out_specs=[
            pl.BlockSpec((BM, 1), lambda i: (i, 0)),
            pl.BlockSpec((1, 1), lambda i: (0, 0)),
        ],
        out_shape=[
            jax.ShapeDtypeStruct((M_TOK, 1), jnp.int32),
            jax.ShapeDtypeStruct((1, 1), jnp.float32),
        ],
    )(x, e, a, b)


def _sc_body(e_hbm, idx3_hbm, idxf_hbm, q_hbm, hist_hbm,
             idx2d_v, idxf_v, rows_v, hist_v, sem):
    wid = lax.axis_index("s") * _NC + lax.axis_index("c")
    base = wid * _BW
    pltpu.sync_copy(idxf_hbm.at[pl.ds(base, _BW)], idxf_v)
    pltpu.sync_copy(idx3_hbm.at[wid], idx2d_v)

    for c in range(_BW // _CH):
        pltpu.async_copy(e_hbm.at[idx2d_v.at[c]], rows_v, sem).wait()
        pltpu.sync_copy(rows_v, q_hbm.at[pl.ds(base + c * _CH, _CH)])

    def _zbody(i, carry):
        hist_v[pl.ds(i * 16, 16)] = jnp.zeros((16,), jnp.float32)
        return carry

    lax.fori_loop(0, N_EMB // 16, _zbody, 0)

    ones16 = jnp.ones((16,), jnp.float32)

    def _sbody(g, carry):
        iv = idxf_v[pl.ds(g * 16, 16)]
        plsc.addupdate_scatter(hist_v, [iv], ones16)
        return carry

    lax.fori_loop(0, _BW // 16, _sbody, 0)

    pltpu.sync_copy(hist_v, hist_hbm.at[wid])


def _sc_gather_hist(e, idx_flat):
    mesh = plsc.VectorSubcoreMesh(core_axis_name="c", subcore_axis_name="s")
    kern = pl.kernel(
        _sc_body,
        mesh=mesh,
        out_type=[
            jax.ShapeDtypeStruct((M_TOK, DIM), jnp.float32),
            jax.ShapeDtypeStruct((_NW, N_EMB), jnp.float32),
        ],
        scratch_types=[
            pltpu.VMEM((_BW // _CH, _CH), jnp.int32),
            pltpu.VMEM((_BW,), jnp.int32),
            pltpu.VMEM((_CH, DIM), jnp.float32),
            pltpu.VMEM((N_EMB,), jnp.float32),
            pltpu.SemaphoreType.DMA,
        ],
    )
    idx3 = idx_flat.reshape(_NW, _BW // _CH, _CH)
    return kern(e, idx3, idx_flat)


def _scalar_body(hist_ref, dsum_ref, loss_ref, perp_ref):
    counts = jnp.sum(hist_ref[...], axis=0, keepdims=True)
    avg = counts * (1.0 / M_TOK)
    ent = avg * jnp.log(avg + 1e-10)
    perp_ref[...] = jnp.exp(-jnp.sum(ent, axis=1, keepdims=True))
    loss_ref[...] = dsum_ref[...] * (1.25 / (M_TOK * DIM))


def _scalars(hist_parts, dsum):
    return pl.pallas_call(
        _scalar_body,
        out_shape=[
            jax.ShapeDtypeStruct((1, 1), jnp.float32),
            jax.ShapeDtypeStruct((1, 1), jnp.float32),
        ],
    )(hist_parts, dsum)


def kernel(inputs, embedding_weight):
    input_shape = inputs.shape
    flat = inputs.reshape(-1, DIM)
    a = jnp.sum(flat ** 2, axis=1, keepdims=True)
    b = jnp.sum(embedding_weight ** 2, axis=1)[None, :]

    idx2, dsum = _dist_argmin(flat, embedding_weight, a, b)
    idx_flat = idx2.reshape(M_TOK)

    quantized, hist_parts = _sc_gather_hist(embedding_weight, idx_flat)
    loss_m, perp_m = _scalars(hist_parts, dsum)

    loss = loss_m[0, 0]
    perplexity = perp_m[0, 0]
    quantized_st = quantized.reshape(input_shape)
    indices = idx_flat.reshape(input_shape[0], -1)
    return (loss, quantized_st, perplexity, indices)


# speedup vs baseline: 1.3401x; 1.3401x over previous
import jax
import jax.numpy as jnp
from jax import lax
from jax.experimental import pallas as pl
from jax.experimental.pallas import tpu as pltpu
from jax.experimental.pallas import tpu_sc as plsc

N_EMB = 8192
DIM = 256
M_TOK = 16384
BM = 512
BN = 2048
GM = M_TOK // BM

_NC = 2
_NS = 16
_NW = _NC * _NS
_BW = M_TOK // _NW
_CH = 128


_B1 = 2736
_B2 = 5472
_RANGES = ((0, _B1), (_B1, _B2), (_B2, N_EMB))


def _bf16_round(x):
    u = lax.bitcast_convert_type(x, jnp.uint32)
    r = (u + jnp.uint32(0x7FFF) + ((u >> jnp.uint32(16)) & jnp.uint32(1))) \
        & jnp.uint32(0xFFFF0000)
    return lax.bitcast_convert_type(r, jnp.float32)


def _argmin_body(x_ref, e_ref, a_ref, b_ref, idx_ref, dsum_ref):
    i = pl.program_id(0)
    x = x_ref[...]
    a = a_ref[...]
    runs = [None, None, None]
    for c in range(N_EMB // BN):
        lo = c * BN
        ec = e_ref[pl.ds(lo, BN), :]
        bc = b_ref[:, pl.ds(lo, BN)]
        m = lax.dot_general(x, ec, (((1,), (1,)), ((), ())),
                            preferred_element_type=jnp.float32)
        d = (a + bc) - 2.0 * m
        col = lax.broadcasted_iota(jnp.int32, (BM, BN), 1) + lo
        for r, (rlo, rhi) in enumerate(_RANGES):
            olo, ohi = max(rlo, lo), min(rhi, lo + BN)
            if olo >= ohi:
                continue
            if olo == lo and ohi == lo + BN:
                dm = d
            else:
                msk = (col >= olo) & (col < ohi)
                dm = jnp.where(msk, d, jnp.float32(jnp.inf))
            cmin = jnp.min(dm, axis=1, keepdims=True)
            cidx = jnp.min(jnp.where(dm == cmin, col, jnp.int32(N_EMB)),
                           axis=1, keepdims=True)
            if runs[r] is None:
                runs[r] = (cmin, cidx)
            else:
                pv, pi = runs[r]
                better = cmin < pv
                runs[r] = (jnp.where(better, cmin, pv),
                           jnp.where(better, cidx, pi))
    (m_a, i_a), (m_b, i_b), (m_c, i_c) = runs
    vq = _bf16_round(m_a)
    idx = i_a
    dsel = m_a
    take = m_b < vq
    idx = jnp.where(take, i_b, idx)
    dsel = jnp.where(take, m_b, dsel)
    vq = jnp.where(take, _bf16_round(m_b), vq)
    take = m_c < vq
    idx = jnp.where(take, i_c, idx)
    dsel = jnp.where(take, m_c, dsel)
    idx_ref[...] = idx

    @pl.when(i == 0)
    def _():
        dsum_ref[...] = jnp.zeros((1, 1), jnp.float32)

    dsum_ref[...] += jnp.sum(dsel, axis=0, keepdims=True)


def _dist_argmin(x, e, a, b):
    return pl.pallas_call(
        _argmin_body,
        grid=(GM,),
        in_specs=[
            pl.BlockSpec((BM, DIM), lambda i: (i, 0)),
            pl.BlockSpec((N_EMB, DIM), lambda i: (0, 0)),
            pl.BlockSpec((BM, 1), lambda i: (i, 0)),
            pl.BlockSpec((1, N_EMB), lambda i: (0, 0)),
        ],
        out_specs=[
            pl.BlockSpec((BM, 1), lambda i: (i, 0)),
            pl.BlockSpec((1, 1), lambda i: (0, 0)),
        ],
        out_shape=[
            jax.ShapeDtypeStruct((M_TOK, 1), jnp.int32),
            jax.ShapeDtypeStruct((1, 1), jnp.float32),
        ],
    )(x, e, a, b)


def _sc_body(e_hbm, idxf_hbm, q_hbm, hist_hbm,
             idxf_v, rows_v, hist_v, sem):
    wid = lax.axis_index("s") * _NC + lax.axis_index("c")
    base = wid * _BW
    pltpu.sync_copy(idxf_hbm.at[pl.ds(base, _BW)], idxf_v)

    for c in range(_BW // _CH):
        pltpu.async_copy(e_hbm.at[idxf_v.at[pl.ds(c * _CH, _CH)]],
                         rows_v, sem).wait()
        pltpu.sync_copy(rows_v, q_hbm.at[pl.ds(base + c * _CH, _CH)])

    def _zbody(i, carry):
        hist_v[pl.ds(i * 16, 16)] = jnp.zeros((16,), jnp.float32)
        return carry

    lax.fori_loop(0, N_EMB // 16, _zbody, 0)

    ones16 = jnp.ones((16,), jnp.float32)

    def _sbody(g, carry):
        iv = idxf_v[pl.ds(g * 16, 16)]
        plsc.addupdate_scatter(hist_v, [iv], ones16)
        return carry

    lax.fori_loop(0, _BW // 16, _sbody, 0)

    pltpu.sync_copy(hist_v, hist_hbm.at[wid])


def _sc_gather_hist(e, idx_flat):
    mesh = plsc.VectorSubcoreMesh(core_axis_name="c", subcore_axis_name="s")
    kern = pl.kernel(
        _sc_body,
        mesh=mesh,
        compiler_params=pltpu.CompilerParams(needs_layout_passes=False),
        out_type=[
            jax.ShapeDtypeStruct((M_TOK, DIM), jnp.float32),
            jax.ShapeDtypeStruct((_NW, N_EMB), jnp.float32),
        ],
        scratch_types=[
            pltpu.VMEM((_BW,), jnp.int32),
            pltpu.VMEM((_CH, DIM), jnp.float32),
            pltpu.VMEM((N_EMB,), jnp.float32),
            pltpu.SemaphoreType.DMA,
        ],
    )
    return kern(e, idx_flat)


def _scalar_body(hist_ref, dsum_ref, loss_ref, perp_ref):
    counts = jnp.sum(hist_ref[...], axis=0, keepdims=True)
    avg = counts * (1.0 / M_TOK)
    ent = avg * jnp.log(avg + 1e-10)
    perp_ref[...] = jnp.exp(-jnp.sum(ent, axis=1, keepdims=True))
    loss_ref[...] = dsum_ref[...] * (1.25 / (M_TOK * DIM))


def _scalars(hist_parts, dsum):
    return pl.pallas_call(
        _scalar_body,
        out_shape=[
            jax.ShapeDtypeStruct((1, 1), jnp.float32),
            jax.ShapeDtypeStruct((1, 1), jnp.float32),
        ],
    )(hist_parts, dsum)


def kernel(inputs, embedding_weight):
    input_shape = inputs.shape
    flat = inputs.reshape(-1, DIM)
    a = jnp.sum(flat ** 2, axis=1, keepdims=True)
    b = jnp.sum(embedding_weight ** 2, axis=1)[None, :]

    idx2, dsum = _dist_argmin(flat, embedding_weight, a, b)
    idx_flat = idx2.reshape(M_TOK)

    quantized, hist_parts = _sc_gather_hist(embedding_weight, idx_flat)
    loss_m, perp_m = _scalars(hist_parts, dsum)

    loss = loss_m[0, 0]
    perplexity = perp_m[0, 0]
    quantized_st = quantized.reshape(input_shape)
    indices = idx_flat.reshape(input_shape[0], -1)
    return (loss, quantized_st, perplexity, indices)


# speedup vs baseline: 1.5828x; 1.1810x over previous
import jax
import jax.numpy as jnp
from jax import lax
from jax.experimental import pallas as pl
from jax.experimental.pallas import tpu as pltpu
from jax.experimental.pallas import tpu_sc as plsc

N_EMB = 8192
DIM = 256
M_TOK = 16384
BM = 512
GM = M_TOK // BM

_NC = 2
_NS = 16
_NW = _NC * _NS
_BW = M_TOK // _NW
_CH = 128


_B1 = 2736
_B2 = 5472
_RANGES = ((0, _B1), (_B1, _B2), (_B2, N_EMB))


def _bf16_round(x):
    u = lax.bitcast_convert_type(x, jnp.uint32)
    r = (u + jnp.uint32(0x7FFF) + ((u >> jnp.uint32(16)) & jnp.uint32(1))) \
        & jnp.uint32(0xFFFF0000)
    return lax.bitcast_convert_type(r, jnp.float32)


def _argmin_body(x_ref, e0_ref, e1_ref, e2_ref, a_ref,
                 b0_ref, b1_ref, b2_ref, idx_ref, dsum_ref):
    i = pl.program_id(0)
    x = x_ref[...]
    a = a_ref[...]
    runs = []
    for (rlo, rhi), e_ref, b_ref in zip(
            _RANGES, (e0_ref, e1_ref, e2_ref), (b0_ref, b1_ref, b2_ref)):
        cn = rhi - rlo
        m = lax.dot_general(x, e_ref[...], (((1,), (1,)), ((), ())),
                            preferred_element_type=jnp.float32)
        d = (a + b_ref[...]) - 2.0 * m
        col = lax.broadcasted_iota(jnp.int32, (BM, cn), 1) + rlo
        cmin = jnp.min(d, axis=1, keepdims=True)
        cidx = jnp.min(jnp.where(d == cmin, col, jnp.int32(N_EMB)),
                       axis=1, keepdims=True)
        runs.append((cmin, cidx))
    (m_a, i_a), (m_b, i_b), (m_c, i_c) = runs
    vq = _bf16_round(m_a)
    idx = i_a
    dsel = m_a
    take = m_b < vq
    idx = jnp.where(take, i_b, idx)
    dsel = jnp.where(take, m_b, dsel)
    vq = jnp.where(take, _bf16_round(m_b), vq)
    take = m_c < vq
    idx = jnp.where(take, i_c, idx)
    dsel = jnp.where(take, m_c, dsel)
    idx_ref[...] = idx

    @pl.when(i == 0)
    def _():
        dsum_ref[...] = jnp.zeros((1, 1), jnp.float32)

    dsum_ref[...] += jnp.sum(dsel, axis=0, keepdims=True)


def _dist_argmin(x, e, a, b):
    e_parts = [e[lo:hi] for lo, hi in _RANGES]
    b_parts = [b[:, lo:hi] for lo, hi in _RANGES]
    e_specs = [pl.BlockSpec((hi - lo, DIM), lambda i: (0, 0))
               for lo, hi in _RANGES]
    b_specs = [pl.BlockSpec((1, hi - lo), lambda i: (0, 0))
               for lo, hi in _RANGES]
    return pl.pallas_call(
        _argmin_body,
        grid=(GM,),
        in_specs=[pl.BlockSpec((BM, DIM), lambda i: (i, 0))] + e_specs
                 + [pl.BlockSpec((BM, 1), lambda i: (i, 0))] + b_specs,
        out_specs=[
            pl.BlockSpec((BM, 1), lambda i: (i, 0)),
            pl.BlockSpec((1, 1), lambda i: (0, 0)),
        ],
        out_shape=[
            jax.ShapeDtypeStruct((M_TOK, 1), jnp.int32),
            jax.ShapeDtypeStruct((1, 1), jnp.float32),
        ],
    )(x, *e_parts, a, *b_parts)


def _sc_body(e_hbm, idxf_hbm, q_hbm, hist_hbm,
             idxf_v, rows_v, hist_v, sem):
    wid = lax.axis_index("s") * _NC + lax.axis_index("c")
    base = wid * _BW
    pltpu.sync_copy(idxf_hbm.at[pl.ds(base, _BW)], idxf_v)

    for c in range(_BW // _CH):
        pltpu.async_copy(e_hbm.at[idxf_v.at[pl.ds(c * _CH, _CH)]],
                         rows_v, sem).wait()
        pltpu.sync_copy(rows_v, q_hbm.at[pl.ds(base + c * _CH, _CH)])

    def _zbody(i, carry):
        hist_v[pl.ds(i * 16, 16)] = jnp.zeros((16,), jnp.float32)
        return carry

    lax.fori_loop(0, N_EMB // 16, _zbody, 0)

    ones16 = jnp.ones((16,), jnp.float32)

    def _sbody(g, carry):
        iv = idxf_v[pl.ds(g * 16, 16)]
        plsc.addupdate_scatter(hist_v, [iv], ones16)
        return carry

    lax.fori_loop(0, _BW // 16, _sbody, 0)

    pltpu.sync_copy(hist_v, hist_hbm.at[wid])


def _sc_gather_hist(e, idx_flat):
    mesh = plsc.VectorSubcoreMesh(core_axis_name="c", subcore_axis_name="s")
    kern = pl.kernel(
        _sc_body,
        mesh=mesh,
        compiler_params=pltpu.CompilerParams(needs_layout_passes=False),
        out_type=[
            jax.ShapeDtypeStruct((M_TOK, DIM), jnp.float32),
            jax.ShapeDtypeStruct((_NW, N_EMB), jnp.float32),
        ],
        scratch_types=[
            pltpu.VMEM((_BW,), jnp.int32),
            pltpu.VMEM((_CH, DIM), jnp.float32),
            pltpu.VMEM((N_EMB,), jnp.float32),
            pltpu.SemaphoreType.DMA,
        ],
    )
    return kern(e, idx_flat)


def _scalar_body(hist_ref, dsum_ref, loss_ref, perp_ref):
    counts = jnp.sum(hist_ref[...], axis=0, keepdims=True)
    avg = counts * (1.0 / M_TOK)
    ent = avg * jnp.log(avg + 1e-10)
    perp_ref[...] = jnp.exp(-jnp.sum(ent, axis=1, keepdims=True))
    loss_ref[...] = dsum_ref[...] * (1.25 / (M_TOK * DIM))


def _scalars(hist_parts, dsum):
    return pl.pallas_call(
        _scalar_body,
        out_shape=[
            jax.ShapeDtypeStruct((1, 1), jnp.float32),
            jax.ShapeDtypeStruct((1, 1), jnp.float32),
        ],
    )(hist_parts, dsum)


def kernel(inputs, embedding_weight):
    input_shape = inputs.shape
    flat = inputs.reshape(-1, DIM)
    a = jnp.sum(flat ** 2, axis=1, keepdims=True)
    b = jnp.sum(embedding_weight ** 2, axis=1)[None, :]

    idx2, dsum = _dist_argmin(flat, embedding_weight, a, b)
    idx_flat = idx2.reshape(M_TOK)

    quantized, hist_parts = _sc_gather_hist(embedding_weight, idx_flat)
    loss_m, perp_m = _scalars(hist_parts, dsum)

    loss = loss_m[0, 0]
    perplexity = perp_m[0, 0]
    quantized_st = quantized.reshape(input_shape)
    indices = idx_flat.reshape(input_shape[0], -1)
    return (loss, quantized_st, perplexity, indices)
